# baseline (speedup 1.0000x reference)
import jax
import jax.numpy as jnp
from jax.experimental import pallas as pl
from jax.experimental.pallas import tpu as pltpu
from jax.experimental.pallas import tpu_sc as plsc

F32 = jnp.float32
BF16 = jnp.bfloat16


def _mm(a, b, dims):
    return jax.lax.dot_general(a.astype(BF16), b.astype(BF16), dims,
                               preferred_element_type=F32)


def _mmf(a, b, dims):
    return jax.lax.dot_general(a, b, dims, preferred_element_type=F32)


def _linf(x, w, b):
    n = x.shape[:-1]
    y = _mmf(x.reshape(-1, x.shape[-1]), w, (((1,), (1,)), ((), ())))
    return y.reshape(*n, w.shape[0]) + b


def _sc_gather(emb, idx_flat, n_idx, width):
    mesh = plsc.VectorSubcoreMesh(core_axis_name="c", subcore_axis_name="s")

    @pl.kernel(
        out_type=jax.ShapeDtypeStruct((n_idx, emb.shape[1]), emb.dtype),
        mesh=mesh,
    )
    def gather_kernel(emb_hbm, i_hbm, o_hbm):
        def body(i_vmem, o_vmem):
            pltpu.sync_copy(emb_hbm.at[i_vmem.at[0]], o_vmem)

        pltpu.emit_pipeline(
            body,
            grid=(n_idx // width,),
            in_specs=[pl.BlockSpec((1, width), index_map=lambda i: (0, i))],
            out_specs=[
                pl.BlockSpec((width, emb.shape[1]), index_map=lambda i: (i, 0))
            ],
            core_axis_name=("c", "s"),
            dimension_semantics=(pltpu.PARALLEL,),
        )(i_hbm, o_hbm)

    return gather_kernel(emb, idx_flat)


def _ln(x, g, b, eps=1e-6):
    mu = jnp.mean(x, axis=-1, keepdims=True)
    xc = x - mu
    var = jnp.mean(xc * xc, axis=-1, keepdims=True)
    return g * xc / jnp.sqrt(var + eps) + b


def _lin(x, w, b):
    n = x.shape[:-1]
    y = _mm(x.reshape(-1, x.shape[-1]), w, (((1,), (1,)), ((), ())))
    return y.reshape(*n, w.shape[0]) + b


def _tc_forward(xg, W_ih, W_hh, bias, qkv_W, qkv_b, o_W, o_b, ln1, ln2,
                ffn_W1, ffn_b1, ffn_W2, ffn_b2, cls_W, cls_b2d,
                block_b=512, chunk=128, interpret=False):
    S, B, E = xg.shape
    NL, G4, _ = W_ih.shape
    HID = G4 // 4
    NTB = qkv_W.shape[0]
    NH = 4
    dk = HID // NH
    OUT = cls_W.shape[0]
    scale = 1.0 / (dk ** 0.5)
    Sp = -(-S // 8) * 8

    def body(x_ref, wih_ref, whh_ref, bias_ref, qW_ref, qb_ref, oW_ref,
             ob_ref, ln1_ref, ln2_ref, w1_ref, b1_ref, w2_ref, b2_ref,
             cw_ref, cb_ref, out_ref, buf):
        Bb = block_b
        wxh = jnp.concatenate(
            [wih_ref[...], whh_ref[...]], axis=2).astype(BF16)
        bias4 = bias_ref[...].reshape(NL, 1, G4)
        lidx = jax.lax.broadcasted_iota(jnp.int32, (NL, 1, 1), 0)

        def step(w, carry):
            h4, c4 = carry
            x_t = x_ref[jnp.minimum(w, S - 1)]
            hb = h4.astype(BF16)
            xin4 = jnp.concatenate([x_t[None], hb[:NL - 1]], axis=0)
            xh = jnp.concatenate([xin4, hb], axis=2)
            g = jax.lax.dot_general(
                    xh, wxh, (((2,), (2,)), ((0,), (0,))),
                    preferred_element_type=F32) + bias4
            i_ = jax.nn.sigmoid(g[:, :, 0:HID])
            f_ = jax.nn.sigmoid(g[:, :, HID:2 * HID])
            gg = jnp.tanh(g[:, :, 2 * HID:3 * HID])
            o_ = jax.nn.sigmoid(g[:, :, 3 * HID:4 * HID])
            cn = f_ * c4 + i_ * gg
            hn = o_ * jnp.tanh(cn)
            valid = (w >= lidx) & (w - lidx <= S - 1)
            c4 = jnp.where(valid, cn, c4)
            h4 = jnp.where(valid, hn, h4)
            buf[jnp.maximum(w - (NL - 1), 0)] = h4[NL - 1].astype(BF16)
            return (h4, c4)

        z = jnp.zeros((NL, Bb, HID), F32)
        jax.lax.fori_loop(0, S + NL - 1, step, (z, z))
        buf[S:] = jnp.zeros((Sp - S, Bb, HID), BF16)

        qpos = jax.lax.broadcasted_iota(jnp.int32, (1, Sp, 1), 1)
        kpos = jax.lax.broadcasted_iota(jnp.int32, (1, 1, Sp), 2)
        didx = jax.lax.broadcasted_iota(jnp.int32, (1, 1, HID), 2)
        ones_ss = jnp.ones((Sp, Sp), F32)
        for c in range(Bb // chunk):
            cs = slice(c * chunk, (c + 1) * chunk)
            X = jnp.transpose(buf[:, cs, :], (1, 0, 2)).astype(F32)
            for t in range(NTB):
                Y = _ln(X, ln1_ref[t, 0], ln1_ref[t, 1])
                qkv = _lin(Y, qW_ref[t], qb_ref[t])
                q = qkv[:, :, 0:HID]
                k = qkv[:, :, HID:2 * HID]
                v = qkv[:, :, 2 * HID:3 * HID]
                ctx = jnp.zeros(v.shape, F32)
                for h in range(NH):
                    hmask = (didx >= h * dk) & (didx < (h + 1) * dk)
                    kh = jnp.where(hmask, k, 0.0)
                    s = _mm(q, kh, (((2,), (2,)), ((0,), (0,)))) * scale
                    s = jnp.where(qpos == S - 1, 0.0, s)
                    e = jnp.where(kpos >= S, 0.0, jnp.exp(s))
                    den = _mm(e.reshape(chunk * Sp, Sp), ones_ss,
                              (((1,), (0,)), ((), ())))
                    a = (e.reshape(chunk * Sp, Sp) / den).reshape(
                        chunk, Sp, Sp)
                    vh = jnp.where(hmask, v, 0.0)
                    ctx = ctx + _mm(a, vh, (((2,), (1,)), ((0,), (0,))))
                X = X + _linf(ctx, oW_ref[t], ob_ref[t])
                Y2 = _ln(X, ln2_ref[t, 0], ln2_ref[t, 1])
                ff = _linf(jax.nn.gelu(_lin(Y2, w1_ref[t], b1_ref[t])),
                           w2_ref[t], b2_ref[t])
                X = X + ff
            xlast = X[:, S - 1, :]
            out_ref[cs, :] = _mmf(
                xlast, cw_ref[...],
                (((1,), (1,)), ((), ()))) + cb_ref[...]

    nblk = B // block_b
    full = lambda shape: pl.BlockSpec(shape, lambda i: tuple(0 for _ in shape))
    in_specs = [
        pl.BlockSpec((S, block_b, E), lambda i: (0, i, 0)),
        full(W_ih.shape), full(W_hh.shape), full(bias.shape),
        full(qkv_W.shape), full(qkv_b.shape), full(o_W.shape),
        full(o_b.shape), full(ln1.shape),
        full(ln2.shape), full(ffn_W1.shape), full(ffn_b1.shape),
        full(ffn_W2.shape), full(ffn_b2.shape), full(cls_W.shape),
        full(cls_b2d.shape),
    ]
    return pl.pallas_call(
        body,
        grid=(nblk,),
        in_specs=in_specs,
        out_specs=pl.BlockSpec((block_b, OUT), lambda i: (i, 0)),
        out_shape=jax.ShapeDtypeStruct((B, OUT), F32),
        scratch_shapes=[
            pltpu.VMEM((Sp, block_b, HID), BF16),
        ],
        compiler_params=pltpu.CompilerParams(
            dimension_semantics=("arbitrary",),
        ),
        interpret=interpret,
    )(xg, W_ih, W_hh, bias, qkv_W, qkv_b, o_W, o_b, ln1, ln2,
      ffn_W1, ffn_b1, ffn_W2, ffn_b2, cls_W, cls_b2d)


def kernel(items, emb, W_ih, W_hh, b_ih, b_hh, attn_W, attn_b, ln1, ln2,
           ffn_W1, ffn_b1, ffn_W2, ffn_b2, cls_W, cls_b):
    B, L = items.shape
    S = L + 1
    E = emb.shape[1]
    cls_id = emb.shape[0] - 1
    seq_tm = jnp.concatenate(
        [items.T.astype(jnp.int32),
         jnp.full((1, B), cls_id, dtype=jnp.int32)], axis=0)
    xg = _sc_gather(emb, seq_tm.reshape(1, S * B), S * B, 128)
    xg = xg.reshape(S, B, E).astype(BF16)
    ntb, _, hid, _ = attn_W.shape
    qkv_W = attn_W[:, :3].reshape(ntb, 3 * hid, hid)
    qkv_b = attn_b[:, :3].reshape(ntb, 3 * hid)
    logits = _tc_forward(
        xg, W_ih, W_hh, b_ih + b_hh, qkv_W, qkv_b, attn_W[:, 3],
        attn_b[:, 3], ln1, ln2,
        ffn_W1, ffn_b1, ffn_W2, ffn_b2, cls_W,
        cls_b.reshape(1, -1))
    return logits


# speedup vs baseline: 1.0224x; 1.0224x over previous
import jax
import jax.numpy as jnp
from jax.experimental import pallas as pl
from jax.experimental.pallas import tpu as pltpu
from jax.experimental.pallas import tpu_sc as plsc

F32 = jnp.float32
BF16 = jnp.bfloat16


def _mm(a, b, dims):
    return jax.lax.dot_general(a.astype(BF16), b.astype(BF16), dims,
                               preferred_element_type=F32)


def _mmf(a, b, dims):
    return jax.lax.dot_general(a, b, dims, preferred_element_type=F32)


def _linf(x, w, b):
    n = x.shape[:-1]
    y = _mmf(x.reshape(-1, x.shape[-1]), w, (((1,), (1,)), ((), ())))
    return y.reshape(*n, w.shape[0]) + b


def _sc_gather(emb, idx_flat, n_idx, width):
    mesh = plsc.VectorSubcoreMesh(core_axis_name="c", subcore_axis_name="s")

    @pl.kernel(
        out_type=jax.ShapeDtypeStruct((n_idx, emb.shape[1]), emb.dtype),
        mesh=mesh,
    )
    def gather_kernel(emb_hbm, i_hbm, o_hbm):
        def body(i_vmem, o_vmem):
            pltpu.sync_copy(emb_hbm.at[i_vmem.at[0]], o_vmem)

        pltpu.emit_pipeline(
            body,
            grid=(n_idx // width,),
            in_specs=[pl.BlockSpec((1, width), index_map=lambda i: (0, i))],
            out_specs=[
                pl.BlockSpec((width, emb.shape[1]), index_map=lambda i: (i, 0))
            ],
            core_axis_name=("c", "s"),
            dimension_semantics=(pltpu.PARALLEL,),
        )(i_hbm, o_hbm)

    return gather_kernel(emb, idx_flat)


def _ln(x, g, b, eps=1e-6):
    mu = jnp.mean(x, axis=-1, keepdims=True)
    xc = x - mu
    var = jnp.mean(xc * xc, axis=-1, keepdims=True)
    return g * xc / jnp.sqrt(var + eps) + b


def _lin(x, w, b):
    n = x.shape[:-1]
    y = _mm(x.reshape(-1, x.shape[-1]), w, (((1,), (1,)), ((), ())))
    return y.reshape(*n, w.shape[0]) + b


def _tc_forward(xg, W_ih, W_hh, bias, qkv_W, qkv_b, o_W, o_b, ln1, ln2,
                ffn_W1, ffn_b1, ffn_W2, ffn_b2, cls_W, cls_b2d,
                block_b=512, chunk=128, interpret=False):
    S, B, E = xg.shape
    NL, G4, _ = W_ih.shape
    HID = G4 // 4
    NTB = qkv_W.shape[0]
    NH = 4
    dk = HID // NH
    OUT = cls_W.shape[0]
    scale = 1.0 / (dk ** 0.5)
    Sp = -(-S // 8) * 8

    def body(x_ref, wih_ref, whh_ref, bias_ref, qW_ref, qb_ref, oW_ref,
             ob_ref, ln1_ref, ln2_ref, w1_ref, b1_ref, w2_ref, b2_ref,
             cw_ref, cb_ref, out_ref, buf):
        Bb = block_b
        wxh = jnp.concatenate(
            [wih_ref[...], whh_ref[...]], axis=2).astype(BF16)
        bias4 = bias_ref[...].reshape(NL, 1, G4)
        lidx = jax.lax.broadcasted_iota(jnp.int32, (NL, 1, 1), 0)

        def step(w, carry):
            h4, c4 = carry
            x_t = x_ref[jnp.minimum(w, S - 1)]
            hb = h4.astype(BF16)
            xin4 = jnp.concatenate([x_t[None].astype(BF16), hb[:NL - 1]],
                                   axis=0)
            xh = jnp.concatenate([xin4, hb], axis=2)
            g = jax.lax.dot_general(
                    xh, wxh, (((2,), (2,)), ((0,), (0,))),
                    preferred_element_type=F32) + bias4
            i_ = jax.nn.sigmoid(g[:, :, 0:HID])
            f_ = jax.nn.sigmoid(g[:, :, HID:2 * HID])
            gg = jnp.tanh(g[:, :, 2 * HID:3 * HID])
            o_ = jax.nn.sigmoid(g[:, :, 3 * HID:4 * HID])
            cn = f_ * c4 + i_ * gg
            hn = o_ * jnp.tanh(cn)
            valid = (w >= lidx) & (w - lidx <= S - 1)
            c4 = jnp.where(valid, cn, c4)
            h4 = jnp.where(valid, hn, h4)
            buf[jnp.maximum(w - (NL - 1), 0)] = h4[NL - 1].astype(BF16)
            return (h4, c4)

        z = jnp.zeros((NL, Bb, HID), F32)
        jax.lax.fori_loop(0, S + NL - 1, step, (z, z))
        buf[S:] = jnp.zeros((Sp - S, Bb, HID), BF16)

        qpos = jax.lax.broadcasted_iota(jnp.int32, (1, Sp, 1), 1)
        kpos = jax.lax.broadcasted_iota(jnp.int32, (1, 1, Sp), 2)
        didx = jax.lax.broadcasted_iota(jnp.int32, (1, 1, HID), 2)
        ones_ss = jnp.ones((Sp, Sp), F32)
        for c in range(Bb // chunk):
            cs = slice(c * chunk, (c + 1) * chunk)
            X = jnp.transpose(buf[:, cs, :], (1, 0, 2)).astype(F32)
            for t in range(NTB):
                Y = _ln(X, ln1_ref[t, 0], ln1_ref[t, 1])
                qkv = _lin(Y, qW_ref[t], qb_ref[t])
                q = qkv[:, :, 0:HID]
                k = qkv[:, :, HID:2 * HID]
                v = qkv[:, :, 2 * HID:3 * HID]
                ctx = jnp.zeros(v.shape, F32)
                for h in range(NH):
                    hmask = (didx >= h * dk) & (didx < (h + 1) * dk)
                    kh = jnp.where(hmask, k, 0.0)
                    s = _mm(q, kh, (((2,), (2,)), ((0,), (0,)))) * scale
                    s = jnp.where(qpos == S - 1, 0.0, s)
                    e = jnp.where(kpos >= S, 0.0, jnp.exp(s))
                    den = _mm(e.reshape(chunk * Sp, Sp), ones_ss,
                              (((1,), (0,)), ((), ())))
                    a = (e.reshape(chunk * Sp, Sp) / den).reshape(
                        chunk, Sp, Sp)
                    vh = jnp.where(hmask, v, 0.0)
                    ctx = ctx + _mm(a, vh, (((2,), (1,)), ((0,), (0,))))
                X = X + _linf(ctx, oW_ref[t], ob_ref[t])
                Y2 = _ln(X, ln2_ref[t, 0], ln2_ref[t, 1])
                ff = _linf(jax.nn.gelu(_lin(Y2, w1_ref[t], b1_ref[t])),
                           w2_ref[t], b2_ref[t])
                X = X + ff
            xlast = X[:, S - 1, :]
            out_ref[cs, :] = _mmf(
                xlast, cw_ref[...],
                (((1,), (1,)), ((), ()))) + cb_ref[...]

    nblk = B // block_b
    full = lambda shape: pl.BlockSpec(shape, lambda i: tuple(0 for _ in shape))
    in_specs = [
        pl.BlockSpec((S, block_b, E), lambda i: (0, i, 0)),
        full(W_ih.shape), full(W_hh.shape), full(bias.shape),
        full(qkv_W.shape), full(qkv_b.shape), full(o_W.shape),
        full(o_b.shape), full(ln1.shape),
        full(ln2.shape), full(ffn_W1.shape), full(ffn_b1.shape),
        full(ffn_W2.shape), full(ffn_b2.shape), full(cls_W.shape),
        full(cls_b2d.shape),
    ]
    return pl.pallas_call(
        body,
        grid=(nblk,),
        in_specs=in_specs,
        out_specs=pl.BlockSpec((block_b, OUT), lambda i: (i, 0)),
        out_shape=jax.ShapeDtypeStruct((B, OUT), F32),
        scratch_shapes=[
            pltpu.VMEM((Sp, block_b, HID), BF16),
        ],
        compiler_params=pltpu.CompilerParams(
            dimension_semantics=("arbitrary",),
        ),
        interpret=interpret,
    )(xg, W_ih, W_hh, bias, qkv_W, qkv_b, o_W, o_b, ln1, ln2,
      ffn_W1, ffn_b1, ffn_W2, ffn_b2, cls_W, cls_b2d)


def kernel(items, emb, W_ih, W_hh, b_ih, b_hh, attn_W, attn_b, ln1, ln2,
           ffn_W1, ffn_b1, ffn_W2, ffn_b2, cls_W, cls_b):
    B, L = items.shape
    S = L + 1
    E = emb.shape[1]
    cls_id = emb.shape[0] - 1
    seq_tm = jnp.concatenate(
        [items.T.astype(jnp.int32),
         jnp.full((1, B), cls_id, dtype=jnp.int32)], axis=0)
    xg = _sc_gather(emb, seq_tm.reshape(1, S * B), S * B, 256)
    xg = xg.reshape(S, B, E)
    ntb, _, hid, _ = attn_W.shape
    qkv_W = attn_W[:, :3].reshape(ntb, 3 * hid, hid)
    qkv_b = attn_b[:, :3].reshape(ntb, 3 * hid)
    logits = _tc_forward(
        xg, W_ih, W_hh, b_ih + b_hh, qkv_W, qkv_b, attn_W[:, 3],
        attn_b[:, 3], ln1, ln2,
        ffn_W1, ffn_b1, ffn_W2, ffn_b2, cls_W,
        cls_b.reshape(1, -1))
    return logits
